# baseline (speedup 1.0000x reference)
import functools

import jax
import jax.numpy as jnp
from jax import lax
from jax.experimental import pallas as pl
from jax.experimental.pallas import tpu as pltpu
from jax.experimental.pallas import tpu_sc as plsc

VOCAB = 100000
EMBED = 128
B, L = 1024, 200
NPG, NBG = 8, 4
NW = NPG * NBG
LW = L // NPG
BW = B // NBG
CH = 128
NCH = LW * BW // CH
PEW = 32
NLANE = 16
NB = 5
PF = 3


@functools.cache
def _build():
    mesh = plsc.VectorSubcoreMesh(core_axis_name="c", subcore_axis_name="s")

    @functools.partial(
        pl.kernel,
        out_type=jax.ShapeDtypeStruct((B * L, EMBED), jnp.float32),
        mesh=mesh,
        scratch_types=[
            pltpu.VMEM((NCH, CH), jnp.int32),
            pltpu.VMEM((NCH, CH), jnp.int32),
            pltpu.VMEM((PEW, EMBED), jnp.float32),
            [pltpu.VMEM((CH, EMBED), jnp.float32) for _ in range(NB)],
            [pltpu.SemaphoreType.DMA for _ in range(NB)],
            [pltpu.SemaphoreType.DMA for _ in range(NB)],
        ],
    )
    def embed(seq_hbm, oidx_hbm, table_hbm, pe_hbm, out_hbm,
              idx_v, oidx_v, pe_v, bufs, gsems, ssems):
        wid = lax.axis_index("s") * 2 + lax.axis_index("c")
        pg = wid // NBG
        l0 = pg * LW
        a0 = (l0 // 8) * 8
        d0 = l0 - a0
        h_idx = pltpu.async_copy(seq_hbm.at[wid], idx_v, gsems[0])
        h_oidx = pltpu.async_copy(oidx_hbm.at[wid], oidx_v, gsems[1])
        h_pe = pltpu.async_copy(pe_hbm.at[pl.ds(a0, PEW)], pe_v, gsems[2])

        def gather(c, slot):
            pltpu.async_copy(table_hbm.at[idx_v.at[c]], bufs[slot], gsems[slot])

        def scatter(c, slot):
            pltpu.async_copy(bufs[slot], out_hbm.at[oidx_v.at[c]], ssems[slot])

        def gwait(slot):
            pltpu.make_async_copy(
                table_hbm.at[pl.ds(0, CH)], bufs[slot], gsems[slot]).wait()

        def swait(slot):
            pltpu.make_async_copy(
                table_hbm.at[pl.ds(0, CH)], bufs[slot], ssems[slot]).wait()

        h_idx.wait()
        for c in range(PF):
            gather(c, c)
        h_oidx.wait()
        h_pe.wait()

        def body(g, _):
            for k in range(NB):
                c = NB * g + k
                gwait(k)
                buf = bufs[k]
                pe_row = [pe_v[d0 + (c // 2), pl.ds(s * NLANE, NLANE)]
                          for s in range(EMBED // NLANE)]

                @plsc.parallel_loop(0, CH, step=1, unroll=2)
                def _row_add(i):
                    for s in range(EMBED // NLANE):
                        sl = pl.ds(s * NLANE, NLANE)
                        buf[i, sl] = buf[i, sl] + pe_row[s]

                slot2 = (k + PF) % NB

                @pl.when(c >= NB - PF)
                def _():
                    swait(slot2)

                @pl.when(c + PF < NCH)
                def _():
                    gather(c + PF, slot2)

                scatter(c, k)
            return 0

        lax.fori_loop(0, NCH // NB, body, 0)
        for s in range(NB - PF):
            swait((NCH + PF + s) % NB)

    return embed


def kernel(seq, token_table, pe):
    seq_r = (
        seq.T.reshape(NPG, LW, NBG, BW)
        .transpose(0, 2, 1, 3)
        .reshape(NW, NCH, CH)
    )
    bb = jnp.arange(B, dtype=jnp.int32)[None, :]
    ll = jnp.arange(L, dtype=jnp.int32)[:, None]
    oidx = (
        (bb * L + ll).reshape(NPG, LW, NBG, BW)
        .transpose(0, 2, 1, 3)
        .reshape(NW, NCH, CH)
    )
    out = _build()(seq_r, oidx, token_table, pe)
    return out.reshape(B, L, EMBED)


# speedup vs baseline: 1.0023x; 1.0023x over previous
import functools

import jax
import jax.numpy as jnp
from jax import lax
from jax.experimental import pallas as pl
from jax.experimental.pallas import tpu as pltpu
from jax.experimental.pallas import tpu_sc as plsc

VOCAB = 100000
EMBED = 128
B, L = 1024, 200
NPG, NBG = 8, 4
NW = NPG * NBG
LW = L // NPG
BW = B // NBG
CH = 128
NCH = LW * BW // CH
PEW = 32
NLANE = 16
NB = 5
PF = 3


@functools.cache
def _build():
    mesh = plsc.VectorSubcoreMesh(core_axis_name="c", subcore_axis_name="s")

    @functools.partial(
        pl.kernel,
        out_type=jax.ShapeDtypeStruct((B * L, EMBED), jnp.float32),
        mesh=mesh,
        scratch_types=[
            pltpu.VMEM((NCH, CH), jnp.int32),
            pltpu.VMEM((NCH, CH), jnp.int32),
            pltpu.VMEM((PEW, EMBED), jnp.float32),
            [pltpu.VMEM((CH, EMBED), jnp.float32) for _ in range(NB)],
            [pltpu.SemaphoreType.DMA for _ in range(NB)],
            [pltpu.SemaphoreType.DMA for _ in range(NB)],
            [pltpu.SemaphoreType.DMA for _ in range(3)],
        ],
    )
    def embed(seq_hbm, oidx_hbm, table_hbm, pe_hbm, out_hbm,
              idx_v, oidx_v, pe_v, bufs, gsems, ssems, stg):
        wid = lax.axis_index("s") * 2 + lax.axis_index("c")
        pg = wid // NBG
        l0 = pg * LW
        a0 = (l0 // 8) * 8
        d0 = l0 - a0
        h_idx = pltpu.async_copy(seq_hbm.at[wid], idx_v, stg[0])
        h_oidx = pltpu.async_copy(oidx_hbm.at[wid], oidx_v, stg[1])
        h_pe = pltpu.async_copy(pe_hbm.at[pl.ds(a0, PEW)], pe_v, stg[2])

        def gather(c, slot):
            pltpu.async_copy(table_hbm.at[idx_v.at[c]], bufs[slot], gsems[slot])

        def scatter(c, slot):
            pltpu.async_copy(bufs[slot], out_hbm.at[oidx_v.at[c]], ssems[slot])

        def gwait(slot):
            pltpu.make_async_copy(
                table_hbm.at[pl.ds(0, CH)], bufs[slot], gsems[slot]).wait()

        def swait(slot):
            pltpu.make_async_copy(
                table_hbm.at[pl.ds(0, CH)], bufs[slot], ssems[slot]).wait()

        h_idx.wait()
        for c in range(PF):
            gather(c, c)
        h_oidx.wait()
        h_pe.wait()

        def body(g, _):
            for k in range(NB):
                c = NB * g + k
                gwait(k)
                buf = bufs[k]
                pe_row = [pe_v[d0 + (c // 2), pl.ds(s * NLANE, NLANE)]
                          for s in range(EMBED // NLANE)]

                @plsc.parallel_loop(0, CH, step=1, unroll=2)
                def _row_add(i):
                    for s in range(EMBED // NLANE):
                        sl = pl.ds(s * NLANE, NLANE)
                        buf[i, sl] = buf[i, sl] + pe_row[s]

                slot2 = (k + PF) % NB

                @pl.when(c >= NB - PF)
                def _():
                    swait(slot2)

                @pl.when(c + PF < NCH)
                def _():
                    gather(c + PF, slot2)

                scatter(c, k)
            return 0

        lax.fori_loop(0, NCH // NB, body, 0)
        for s in range(NB - PF):
            swait((NCH + PF + s) % NB)

    return embed


def kernel(seq, token_table, pe):
    seq_r = (
        seq.T.reshape(NPG, LW, NBG, BW)
        .transpose(0, 2, 1, 3)
        .reshape(NW, NCH, CH)
    )
    bb = jnp.arange(B, dtype=jnp.int32)[None, :]
    ll = jnp.arange(L, dtype=jnp.int32)[:, None]
    oidx = (
        (bb * L + ll).reshape(NPG, LW, NBG, BW)
        .transpose(0, 2, 1, 3)
        .reshape(NW, NCH, CH)
    )
    out = _build()(seq_r, oidx, token_table, pe)
    return out.reshape(B, L, EMBED)
